# baseline (speedup 1.0000x reference)
import functools

import jax
import jax.numpy as jnp
from jax import lax
from jax.experimental import pallas as pl
from jax.experimental.pallas import tpu as pltpu
from jax.experimental.pallas import tpu_sc as plsc

_NC = 2
_NS = 16
_NW = _NC * _NS

_WCHUNK = 400
_GCHUNK = 128


def _mesh():
    return plsc.VectorSubcoreMesh(core_axis_name="c", subcore_axis_name="s")


@functools.lru_cache(maxsize=None)
def _make_widen(V, D):
    n_chunks = V // _WCHUNK
    per_tile = -(-n_chunks // _NW)

    @functools.partial(
        pl.kernel,
        out_type=jax.ShapeDtypeStruct((V, 128), jnp.float32),
        mesh=_mesh(),
        scratch_types=[
            pltpu.VMEM((_WCHUNK, D), jnp.float32),
            pltpu.VMEM((_WCHUNK, 128), jnp.float32),
            pltpu.SemaphoreType.DMA,
            pltpu.SemaphoreType.DMA,
        ],
        compiler_params=pltpu.CompilerParams(use_tc_tiling_on_sc=True),
    )
    def widen_kernel(w_hbm, out_hbm, a_v, b_v, sem_i, sem_o):
        wid = lax.axis_index("s") * _NC + lax.axis_index("c")

        def body(c, carry):
            g = c * _NW + wid

            @pl.when(g < n_chunks)
            def _():
                r0 = g * _WCHUNK
                pltpu.async_copy(w_hbm.at[pl.ds(r0, _WCHUNK), :],
                                 a_v, sem_i).wait()

                def rows(r8, carry2):
                    for r in range(8):
                        for k in range(D // 16):
                            b_v[r8 * 8 + r, pl.ds(16 * k, 16)] = (
                                a_v[r8 * 8 + r, pl.ds(16 * k, 16)])
                    return carry2

                lax.fori_loop(0, _WCHUNK // 8, rows, 0)
                pltpu.async_copy(b_v, out_hbm.at[pl.ds(r0, _WCHUNK), :],
                                 sem_o).wait()
            return carry

        lax.fori_loop(0, per_tile, body, 0)

    return widen_kernel


@functools.lru_cache(maxsize=None)
def _make_gather(B, D):
    b_per_w = B // _NW
    n_chunks = b_per_w // _GCHUNK
    assert n_chunks * _GCHUNK == b_per_w and n_chunks % 2 == 0

    @functools.partial(
        pl.kernel,
        out_type=jax.ShapeDtypeStruct((B, D), jnp.float32),
        mesh=_mesh(),
        scratch_types=[
            pltpu.VMEM((_GCHUNK,), jnp.int32),
            pltpu.VMEM((_GCHUNK,), jnp.int32),
            pltpu.VMEM((_GCHUNK, 128), jnp.float32),
            pltpu.VMEM((_GCHUNK, 128), jnp.float32),
            pltpu.VMEM((_GCHUNK, D), jnp.float32),
            pltpu.VMEM((_GCHUNK, D), jnp.float32),
            pltpu.SemaphoreType.DMA,
            pltpu.SemaphoreType.DMA,
            pltpu.SemaphoreType.DMA,
            pltpu.SemaphoreType.DMA,
            pltpu.SemaphoreType.DMA,
            pltpu.SemaphoreType.DMA,
        ],
        compiler_params=pltpu.CompilerParams(use_tc_tiling_on_sc=True),
    )
    def gather_kernel(idx_hbm, tab_hbm, out_hbm, idx_v0, idx_v1,
                      rows_v0, rows_v1, obuf0, obuf1,
                      si0, si1, sg0, sg1, ss0, ss1):
        idx_v = (idx_v0, idx_v1)
        rows_v = (rows_v0, rows_v1)
        obuf = (obuf0, obuf1)

        def compact(b):
            def rows(r8, carry2):
                for r in range(8):
                    for k in range(D // 16):
                        obuf[b][r8 * 8 + r, pl.ds(16 * k, 16)] = (
                            rows_v[b][r8 * 8 + r, pl.ds(16 * k, 16)])
                return carry2
            lax.fori_loop(0, _GCHUNK // 8, rows, 0)
        wid = lax.axis_index("s") * _NC + lax.axis_index("c")
        base_w = wid * b_per_w
        sem_i = (si0, si1)
        sem_g = (sg0, sg1)
        sem_s = (ss0, ss1)

        def load_idx(g, b):
            pltpu.async_copy(
                idx_hbm.at[pl.ds(base_w + g * _GCHUNK, _GCHUNK)],
                idx_v[b], sem_i[b])

        def gather(b):
            pltpu.async_copy(tab_hbm.at[idx_v[b]], rows_v[b], sem_g[b])

        def store(g, b):
            pltpu.async_copy(
                obuf[b],
                out_hbm.at[pl.ds(base_w + g * _GCHUNK, _GCHUNK), :],
                sem_s[b])

        load_idx(0, 0)
        load_idx(1, 1)
        pltpu.make_async_copy(idx_hbm.at[pl.ds(0, _GCHUNK)],
                              idx_v[0], sem_i[0]).wait()
        gather(0)
        pltpu.make_async_copy(idx_hbm.at[pl.ds(0, _GCHUNK)],
                              idx_v[1], sem_i[1]).wait()
        gather(1)
        pltpu.make_async_copy(tab_hbm.at[idx_v[0]],
                              rows_v[0], sem_g[0]).wait()
        compact(0)
        store(0, 0)
        load_idx(2, 0)
        pltpu.make_async_copy(tab_hbm.at[idx_v[1]],
                              rows_v[1], sem_g[1]).wait()
        compact(1)
        store(1, 1)
        load_idx(3, 1)

        def outer(o, carry):
            for b in range(2):
                g = o * 2 + b
                pltpu.make_async_copy(
                    obuf[b], out_hbm.at[pl.ds(0, _GCHUNK), :],
                    sem_s[b]).wait()
                pltpu.make_async_copy(
                    idx_hbm.at[pl.ds(0, _GCHUNK)], idx_v[b],
                    sem_i[b]).wait()
                gather(b)
                pltpu.make_async_copy(
                    tab_hbm.at[idx_v[b]], rows_v[b], sem_g[b]).wait()
                compact(b)
                store(g, b)
                load_idx(g + 2, b)
            return carry

        lax.fori_loop(1, n_chunks // 2 - 1, outer, 0)

        for b in range(2):
            pltpu.make_async_copy(
                obuf[b], out_hbm.at[pl.ds(0, _GCHUNK), :], sem_s[b]).wait()
            pltpu.make_async_copy(
                idx_hbm.at[pl.ds(0, _GCHUNK)], idx_v[b], sem_i[b]).wait()
            gather(b)
        for b in range(2):
            g = n_chunks - 2 + b
            pltpu.make_async_copy(
                tab_hbm.at[idx_v[b]], rows_v[b], sem_g[b]).wait()
            compact(b)
            store(g, b)
        for b in range(2):
            pltpu.make_async_copy(
                obuf[b], out_hbm.at[pl.ds(0, _GCHUNK), :], sem_s[b]).wait()

    return gather_kernel


def kernel(x, weight):
    shape = x.shape
    B = x.size
    V, D = weight.shape
    flat_idx = jnp.reshape(x.astype(jnp.int32), (B,))
    tab = _make_widen(V, D)(weight)
    out = _make_gather(B, D)(flat_idx, tab)
    return jnp.reshape(out, shape + (D,))


# speedup vs baseline: 1.1026x; 1.1026x over previous
import functools

import jax
import jax.numpy as jnp
from jax import lax
from jax.experimental import pallas as pl
from jax.experimental.pallas import tpu as pltpu
from jax.experimental.pallas import tpu_sc as plsc

_NC = 2
_NS = 16
_NW = _NC * _NS


@functools.lru_cache(maxsize=None)
def _make_gather(NB, NP, D):
    rows_per_tile = NB // _NW
    B = NB * NP
    mesh = plsc.VectorSubcoreMesh(core_axis_name="c", subcore_axis_name="s")

    @functools.partial(
        pl.kernel,
        out_type=jax.ShapeDtypeStruct((NB, NP, D), jnp.float32),
        mesh=mesh,
        scratch_types=[
            pltpu.VMEM((NP,), jnp.int32),
            pltpu.VMEM((NP,), jnp.int32),
            pltpu.VMEM((NP, D), jnp.float32),
            pltpu.VMEM((NP, D), jnp.float32),
            pltpu.SemaphoreType.DMA,
            pltpu.SemaphoreType.DMA,
            pltpu.SemaphoreType.DMA,
            pltpu.SemaphoreType.DMA,
            pltpu.SemaphoreType.DMA,
            pltpu.SemaphoreType.DMA,
        ],
        compiler_params=pltpu.CompilerParams(use_tc_tiling_on_sc=False),
    )
    def gather_kernel(idx_hbm, table_hbm, out_hbm, idx_v0, idx_v1,
                      rows_v0, rows_v1, si0, si1, sg0, sg1, ss0, ss1):
        idx_v = (idx_v0, idx_v1)
        rows_v = (rows_v0, rows_v1)
        wid = lax.axis_index("s") * _NC + lax.axis_index("c")
        row0 = wid * rows_per_tile

        sem_i = (si0, si1)
        sem_g = (sg0, sg1)
        sem_s = (ss0, ss1)

        def load_idx(g, b):
            pltpu.async_copy(
                idx_hbm.at[pl.ds((row0 + g) * NP, NP)], idx_v[b], sem_i[b])

        def gather(b):
            pltpu.async_copy(table_hbm.at[idx_v[b]], rows_v[b], sem_g[b])

        def store(g, b):
            pltpu.async_copy(rows_v[b], out_hbm.at[row0 + g], sem_s[b])

        load_idx(0, 0)
        load_idx(1, 1)
        pltpu.make_async_copy(idx_hbm.at[pl.ds(0, NP)],
                              idx_v[0], sem_i[0]).wait()
        gather(0)
        pltpu.make_async_copy(idx_hbm.at[pl.ds(0, NP)],
                              idx_v[1], sem_i[1]).wait()
        gather(1)
        pltpu.make_async_copy(table_hbm.at[idx_v[0]],
                              rows_v[0], sem_g[0]).wait()
        store(0, 0)
        load_idx(2, 0)
        pltpu.make_async_copy(table_hbm.at[idx_v[1]],
                              rows_v[1], sem_g[1]).wait()
        store(1, 1)
        load_idx(3, 1)

        def outer(o, carry):
            for b in range(2):
                g = o * 2 + b
                pltpu.make_async_copy(
                    rows_v[b], out_hbm.at[row0], sem_s[b]).wait()
                pltpu.make_async_copy(
                    idx_hbm.at[pl.ds(0, NP)], idx_v[b], sem_i[b]).wait()
                gather(b)
                pltpu.make_async_copy(
                    table_hbm.at[idx_v[b]], rows_v[b], sem_g[b]).wait()
                store(g, b)
                load_idx(g + 2, b)
            return carry

        lax.fori_loop(1, rows_per_tile // 2 - 1, outer, 0)

        for b in range(2):
            pltpu.make_async_copy(
                rows_v[b], out_hbm.at[row0], sem_s[b]).wait()
            pltpu.make_async_copy(
                idx_hbm.at[pl.ds(0, NP)], idx_v[b], sem_i[b]).wait()
            gather(b)
        for b in range(2):
            g = rows_per_tile - 2 + b
            pltpu.make_async_copy(
                table_hbm.at[idx_v[b]], rows_v[b], sem_g[b]).wait()
            store(g, b)
        for b in range(2):
            pltpu.make_async_copy(
                rows_v[b], out_hbm.at[row0], sem_s[b]).wait()

    return gather_kernel


def kernel(x, weight):
    NB, NP = x.shape
    D = weight.shape[1]
    flat_idx = jnp.reshape(x.astype(jnp.int32), (NB * NP,))
    return _make_gather(NB, NP, D)(flat_idx, weight)


# speedup vs baseline: 1.2300x; 1.1156x over previous
import functools

import jax
import jax.numpy as jnp
from jax import lax
from jax.experimental import pallas as pl
from jax.experimental.pallas import tpu as pltpu
from jax.experimental.pallas import tpu_sc as plsc

_NC = 2
_NS = 16
_NW = _NC * _NS

_GCHUNK = 128


@functools.lru_cache(maxsize=None)
def _make_gather(B, D):
    b_per_w = B // _NW
    n_chunks = b_per_w // _GCHUNK
    assert n_chunks * _GCHUNK == b_per_w and n_chunks % 2 == 0
    mesh = plsc.VectorSubcoreMesh(core_axis_name="c", subcore_axis_name="s")

    @functools.partial(
        pl.kernel,
        out_type=jax.ShapeDtypeStruct((B, D), jnp.float32),
        mesh=mesh,
        scratch_types=[
            pltpu.VMEM((_GCHUNK,), jnp.int32),
            pltpu.VMEM((_GCHUNK,), jnp.int32),
            pltpu.VMEM((_GCHUNK, 128), jnp.float32),
            pltpu.VMEM((_GCHUNK, 128), jnp.float32),
            pltpu.VMEM((_GCHUNK, D), jnp.float32),
            pltpu.VMEM((_GCHUNK, D), jnp.float32),
            pltpu.SemaphoreType.DMA,
            pltpu.SemaphoreType.DMA,
            pltpu.SemaphoreType.DMA,
            pltpu.SemaphoreType.DMA,
            pltpu.SemaphoreType.DMA,
            pltpu.SemaphoreType.DMA,
        ],
        compiler_params=pltpu.CompilerParams(use_tc_tiling_on_sc=True),
    )
    def gather_kernel(idx_hbm, tab_hbm, out_hbm, idx_v0, idx_v1,
                      rows_v0, rows_v1, obuf0, obuf1,
                      si0, si1, sg0, sg1, ss0, ss1):
        idx_v = (idx_v0, idx_v1)
        rows_v = (rows_v0, rows_v1)
        obuf = (obuf0, obuf1)
        wid = lax.axis_index("s") * _NC + lax.axis_index("c")
        base_w = wid * b_per_w
        sem_i = (si0, si1)
        sem_g = (sg0, sg1)
        sem_s = (ss0, ss1)

        def compact(b):
            def rows(r8, carry2):
                for r in range(8):
                    for k in range(D // 16):
                        obuf[b][r8 * 8 + r, pl.ds(16 * k, 16)] = (
                            rows_v[b][r8 * 8 + r, pl.ds(16 * k, 16)])
                return carry2
            lax.fori_loop(0, _GCHUNK // 8, rows, 0)

        def load_idx(g, b):
            pltpu.async_copy(
                idx_hbm.at[pl.ds(base_w + g * _GCHUNK, _GCHUNK)],
                idx_v[b], sem_i[b])

        def gather(b):
            pltpu.async_copy(tab_hbm.at[idx_v[b]], rows_v[b], sem_g[b])

        def store(g, b):
            pltpu.async_copy(
                obuf[b],
                out_hbm.at[pl.ds(base_w + g * _GCHUNK, _GCHUNK), :],
                sem_s[b])

        load_idx(0, 0)
        load_idx(1, 1)
        pltpu.make_async_copy(idx_hbm.at[pl.ds(0, _GCHUNK)],
                              idx_v[0], sem_i[0]).wait()
        gather(0)
        pltpu.make_async_copy(idx_hbm.at[pl.ds(0, _GCHUNK)],
                              idx_v[1], sem_i[1]).wait()
        gather(1)
        pltpu.make_async_copy(tab_hbm.at[idx_v[0]],
                              rows_v[0], sem_g[0]).wait()
        compact(0)
        store(0, 0)
        load_idx(2, 0)
        pltpu.make_async_copy(tab_hbm.at[idx_v[1]],
                              rows_v[1], sem_g[1]).wait()
        compact(1)
        store(1, 1)
        load_idx(3, 1)

        def outer(o, carry):
            for b in range(2):
                g = o * 2 + b
                pltpu.make_async_copy(
                    obuf[b], out_hbm.at[pl.ds(0, _GCHUNK), :],
                    sem_s[b]).wait()
                pltpu.make_async_copy(
                    idx_hbm.at[pl.ds(0, _GCHUNK)], idx_v[b],
                    sem_i[b]).wait()
                gather(b)
                pltpu.make_async_copy(
                    tab_hbm.at[idx_v[b]], rows_v[b], sem_g[b]).wait()
                compact(b)
                store(g, b)
                load_idx(g + 2, b)
            return carry

        lax.fori_loop(1, n_chunks // 2 - 1, outer, 0)

        for b in range(2):
            pltpu.make_async_copy(
                obuf[b], out_hbm.at[pl.ds(0, _GCHUNK), :], sem_s[b]).wait()
            pltpu.make_async_copy(
                idx_hbm.at[pl.ds(0, _GCHUNK)], idx_v[b], sem_i[b]).wait()
            gather(b)
        for b in range(2):
            g = n_chunks - 2 + b
            pltpu.make_async_copy(
                tab_hbm.at[idx_v[b]], rows_v[b], sem_g[b]).wait()
            compact(b)
            store(g, b)
        for b in range(2):
            pltpu.make_async_copy(
                obuf[b], out_hbm.at[pl.ds(0, _GCHUNK), :], sem_s[b]).wait()

    return gather_kernel


def kernel(x, weight):
    shape = x.shape
    B = x.size
    V, D = weight.shape
    flat_idx = jnp.reshape(x.astype(jnp.int32), (B,))
    tab = jnp.pad(weight, ((0, 0), (0, 128 - D)))
    out = _make_gather(B, D)(flat_idx, tab)
    return jnp.reshape(out, shape + (D,))


# speedup vs baseline: 1.3906x; 1.1306x over previous
import functools

import jax
import jax.numpy as jnp
from jax import lax
from jax.experimental import pallas as pl
from jax.experimental.pallas import tpu as pltpu
from jax.experimental.pallas import tpu_sc as plsc

_NC = 2
_NS = 16
_NW = _NC * _NS

_GCHUNK = 256


@functools.lru_cache(maxsize=None)
def _make_gather(B, D):
    b_per_w = B // _NW
    n_chunks = b_per_w // _GCHUNK
    assert n_chunks * _GCHUNK == b_per_w and n_chunks >= 4
    mesh = plsc.VectorSubcoreMesh(core_axis_name="c", subcore_axis_name="s")

    @functools.partial(
        pl.kernel,
        out_type=jax.ShapeDtypeStruct((B, D), jnp.float32),
        mesh=mesh,
        scratch_types=[
            pltpu.VMEM((b_per_w,), jnp.int32),
            pltpu.VMEM((_GCHUNK, 128), jnp.float32),
            pltpu.VMEM((_GCHUNK, 128), jnp.float32),
            pltpu.VMEM((_GCHUNK, D), jnp.float32),
            pltpu.SemaphoreType.DMA,
            pltpu.SemaphoreType.DMA,
            pltpu.SemaphoreType.DMA,
            pltpu.SemaphoreType.DMA,
        ],
        compiler_params=pltpu.CompilerParams(use_tc_tiling_on_sc=True),
    )
    def gather_kernel(idx_hbm, tab_hbm, out_hbm, idx_v, rows_v0, rows_v1,
                      obuf, sem_i, sg0, sg1, sem_s):
        rows_v = (rows_v0, rows_v1)
        sem_g = (sg0, sg1)
        wid = lax.axis_index("s") * _NC + lax.axis_index("c")
        base_w = wid * b_per_w

        def gather(g, b):
            pltpu.async_copy(
                tab_hbm.at[idx_v.at[pl.ds(g * _GCHUNK, _GCHUNK)]],
                rows_v[b], sem_g[b])

        def wait_gather(b):
            pltpu.make_async_copy(
                tab_hbm.at[idx_v.at[pl.ds(0, _GCHUNK)]],
                rows_v[b], sem_g[b]).wait()

        def compact(b):
            def rows(r8, carry2):
                for r in range(8):
                    for k in range(D // 16):
                        obuf[r8 * 8 + r, pl.ds(16 * k, 16)] = (
                            rows_v[b][r8 * 8 + r, pl.ds(16 * k, 16)])
                return carry2
            lax.fori_loop(0, _GCHUNK // 8, rows, 0)

        def store(g):
            pltpu.async_copy(
                obuf,
                out_hbm.at[pl.ds(base_w + g * _GCHUNK, _GCHUNK), :],
                sem_s)

        def wait_store():
            pltpu.make_async_copy(
                obuf, out_hbm.at[pl.ds(0, _GCHUNK), :], sem_s).wait()

        pltpu.async_copy(idx_hbm.at[pl.ds(base_w, b_per_w)], idx_v, sem_i)
        pltpu.make_async_copy(idx_hbm.at[pl.ds(0, b_per_w)],
                              idx_v, sem_i).wait()
        gather(0, 0)

        def body(g, carry):
            b = lax.rem(g, 2)

            @pl.when(b == 0)
            def _():
                wait_gather(0)

                @pl.when(g + 1 < n_chunks)
                def _():
                    gather(g + 1, 1)

                @pl.when(g >= 1)
                def _():
                    wait_store()
                compact(0)
                store(g)

            @pl.when(b == 1)
            def _():
                wait_gather(1)

                @pl.when(g + 1 < n_chunks)
                def _():
                    gather(g + 1, 0)
                wait_store()
                compact(1)
                store(g)
            return carry

        lax.fori_loop(0, n_chunks, body, 0)
        wait_store()

    return gather_kernel


def kernel(x, weight):
    shape = x.shape
    B = x.size
    V, D = weight.shape
    flat_idx = jnp.reshape(x.astype(jnp.int32), (B,))
    tab = jnp.pad(weight, ((0, 0), (0, 128 - D)))
    out = _make_gather(B, D)(flat_idx, tab)
    return jnp.reshape(out, shape + (D,))
